# baseline (speedup 1.0000x reference)
import functools

import jax
import jax.numpy as jnp
import jax.random as jr
from jax import lax
from jax.experimental import pallas as pl
from jax.experimental.pallas import tpu as pltpu
from jax.experimental.pallas import tpu_sc as plsc

_MAX_FLIPS = 65536
_N = 16777216
_NB = 65536
_NC = 2
_NS = 16
_NW = _NC * _NS
_C = _N // _NW
_WE = 8192
_NWIN = _C // _WE
_ROWS = _WE // 128

_PER_W = _MAX_FLIPS // _NW
_CHUNK = 128
_NCHUNK = _PER_W // _CHUNK

_SC_PARAMS = dict(
    mesh=plsc.VectorSubcoreMesh(core_axis_name="c", subcore_axis_name="s"),
    compiler_params=pltpu.CompilerParams(needs_layout_passes=False),
)


def _wid():
  return lax.axis_index("s") * _NC + lax.axis_index("c")


def _digit(k, hi):
  return lax.shift_right_logical(k, 16) if hi else k & jnp.int32(0xFFFF)


def _hist_body(keys_hbm, hist_out, hist_v, kwin, sem, *, hi):
  del sem
  base = _wid() * _C
  z = jnp.zeros((16,), jnp.int32)

  def zero(t, c):
    hist_v[pl.ds(t * 16, 16)] = z
    return c

  lax.fori_loop(0, _NB // 16, zero, 0)

  def win(w, c):
    pltpu.sync_copy(keys_hbm.at[pl.ds(base + w * _WE, _WE)], kwin)

    def vreg(t, cc):
      d = _digit(kwin[pl.ds(t * 16, 16)], hi)
      cnt, lastm = plsc.scan_count(d)
      plsc.addupdate_scatter(hist_v, [d], cnt, mask=lastm)
      return cc

    lax.fori_loop(0, _WE // 16, vreg, c)
    return c

  lax.fori_loop(0, _NWIN, win, 0)
  pltpu.sync_copy(hist_v, hist_out.at[_wid()])


def _perm_vreg_compute(t, kwin, vwin, cur_v, kstage, vstage, dest, hi,
                       write_keys):
  s = pl.ds(t * 16, 16)
  k = kwin[s]
  v = vwin[s]
  d = _digit(k, hi)
  cnt, lastm = plsc.scan_count(d)
  bse = plsc.load_gather(cur_v, [d])
  dst = bse + cnt - 1
  plsc.addupdate_scatter(cur_v, [d], cnt, mask=lastm)
  if write_keys:
    kstage[s] = k
  vstage[s] = v
  dest[s] = dst


def _perm0_body(keys_hbm, vals_hbm, off_hbm, kout_hbm, vout_hbm,
                cur_v, kwin, vwin, kstage, vstage, dest, sem):
  base = _wid() * _C
  pltpu.sync_copy(off_hbm.at[_wid()], cur_v)

  def win(w, c):
    pltpu.sync_copy(keys_hbm.at[pl.ds(base + w * _WE, _WE)], kwin)
    pltpu.sync_copy(vals_hbm.at[pl.ds(base + w * _WE, _WE)], vwin)

    def vreg(t, cc):
      _perm_vreg_compute(t, kwin, vwin, cur_v, kstage, vstage, dest,
                         hi=False, write_keys=True)
      return cc

    lax.fori_loop(0, _WE // 16, vreg, c)
    kc = pltpu.async_copy(kstage, kout_hbm.at[dest], sem)
    vc = pltpu.async_copy(vstage, vout_hbm.at[dest], sem)
    kc.wait()
    vc.wait()
    return c

  lax.fori_loop(0, _NWIN, win, 0)


def _perm1_body(keys_hbm, vals_hbm, off_hbm, vout_hbm,
                cur_v, kwin, vwin, vstage, dest, sem):
  base = _wid() * _C
  pltpu.sync_copy(off_hbm.at[_wid()], cur_v)

  def win(w, c):
    pltpu.sync_copy(keys_hbm.at[pl.ds(base + w * _WE, _WE)], kwin)
    pltpu.sync_copy(vals_hbm.at[pl.ds(base + w * _WE, _WE)], vwin)

    def vreg(t, cc):
      _perm_vreg_compute(t, kwin, vwin, cur_v, None, vstage, dest,
                         hi=True, write_keys=False)
      return cc

    lax.fori_loop(0, _WE // 16, vreg, c)
    pltpu.async_copy(vstage, vout_hbm.at[dest], sem).wait()
    return c

  lax.fori_loop(0, _NWIN, win, 0)


@functools.cache
def _get_sort_kernels():
  hist_scratch = [
      pltpu.VMEM((_NB,), jnp.int32),
      pltpu.VMEM((_WE,), jnp.int32),
      pltpu.SemaphoreType.DMA,
  ]
  hist0 = pl.kernel(
      functools.partial(_hist_body, hi=False),
      out_type=jax.ShapeDtypeStruct((_NW, _NB), jnp.int32),
      scratch_types=list(hist_scratch),
      **_SC_PARAMS,
  )
  hist1 = pl.kernel(
      functools.partial(_hist_body, hi=True),
      out_type=jax.ShapeDtypeStruct((_NW, _NB), jnp.int32),
      scratch_types=list(hist_scratch),
      **_SC_PARAMS,
  )
  perm0 = pl.kernel(
      _perm0_body,
      out_type=(jax.ShapeDtypeStruct((_N,), jnp.int32),
                jax.ShapeDtypeStruct((_N,), jnp.int32)),
      scratch_types=[
          pltpu.VMEM((_NB,), jnp.int32),
          pltpu.VMEM((_WE,), jnp.int32),
          pltpu.VMEM((_WE,), jnp.int32),
          pltpu.VMEM((_WE,), jnp.int32),
          pltpu.VMEM((_WE,), jnp.int32),
          pltpu.VMEM((_WE,), jnp.int32),
          pltpu.SemaphoreType.DMA,
      ],
      **_SC_PARAMS,
  )
  perm1 = pl.kernel(
      _perm1_body,
      out_type=jax.ShapeDtypeStruct((_N,), jnp.int32),
      scratch_types=[
          pltpu.VMEM((_NB,), jnp.int32),
          pltpu.VMEM((_WE,), jnp.int32),
          pltpu.VMEM((_WE,), jnp.int32),
          pltpu.VMEM((_WE,), jnp.int32),
          pltpu.VMEM((_WE,), jnp.int32),
          pltpu.SemaphoreType.DMA,
      ],
      **_SC_PARAMS,
  )
  return hist0, hist1, perm0, perm1


def _offsets(hist):
  total = hist.sum(axis=0, dtype=jnp.int32)
  excl_d = jnp.cumsum(total, dtype=jnp.int32) - total
  excl_w = jnp.cumsum(hist, axis=0, dtype=jnp.int32) - hist
  return excl_d[None, :] + excl_w


def _sc_sort(keys, vals):
  hist0, hist1, perm0, perm1 = _get_sort_kernels()
  h0 = hist0(keys)
  k1, v1 = perm0(keys, vals, _offsets(h0))
  h1 = hist1(k1)
  return perm1(k1, v1, _offsets(h1))


def _flip_body(idx_hbm, flip_hbm, x_ref, idx_v, val_v, flip_v, sem):
  wid = _wid()
  pltpu.sync_copy(idx_hbm.at[wid], idx_v)
  pltpu.sync_copy(flip_hbm.at[wid], flip_v)

  gathers = [
      pltpu.async_copy(x_ref.at[idx_v.at[j]],
                       val_v.at[pl.ds(j * _CHUNK, _CHUNK)], sem)
      for j in range(_NCHUNK)
  ]
  for g in gathers:
    g.wait()

  def body(t, carry):
    s = pl.ds(t * 16, 16)
    val_v[s] = val_v[s] * flip_v[s]
    return carry

  lax.fori_loop(0, _PER_W // 16, body, 0)

  scatters = [
      pltpu.async_copy(val_v.at[pl.ds(j * _CHUNK, _CHUNK)],
                       x_ref.at[idx_v.at[j]], sem)
      for j in range(_NCHUNK)
  ]
  for s in scatters:
    s.wait()


@functools.cache
def _get_flip_kernel():
  return pl.kernel(
      _flip_body,
      scratch_types=[
          pltpu.VMEM((_NCHUNK, _CHUNK), jnp.int32),
          pltpu.VMEM((_PER_W,), jnp.float32),
          pltpu.VMEM((_PER_W,), jnp.float32),
          pltpu.SemaphoreType.DMA,
      ],
      **_SC_PARAMS,
  )


def kernel(x, key):
  shape = x.shape
  xf = x.ravel()
  key1, key2 = jr.split(key, 2)
  vals = jnp.array([-1, 1], dtype=xf.dtype)
  flip = jr.choice(key2, vals, (_MAX_FLIPS,))

  perm = jnp.arange(_N, dtype=jnp.int32)
  k = key1
  for _ in range(3):
    k, sk = jr.split(k)
    bits = lax.bitcast_convert_type(jr.bits(sk, (_N,), jnp.uint32), jnp.int32)
    perm = _sc_sort(bits, perm)
  i = perm[:_MAX_FLIPS]

  idx3 = i.reshape(_NW, _NCHUNK, _CHUNK)
  flip2 = flip.reshape(_NW, _PER_W)
  ref = jax.new_ref(xf)
  _get_flip_kernel()(idx3, flip2, ref)
  return ref[...].reshape(shape)


# speedup vs baseline: 3.2752x; 3.2752x over previous
import functools

import jax
import jax.numpy as jnp
import jax.random as jr
from jax import lax
from jax.experimental import pallas as pl
from jax.experimental.pallas import tpu as pltpu
from jax.experimental.pallas import tpu_sc as plsc

_MAX_FLIPS = 65536
_N = 16777216
_NB = 65536
_NC = 2
_NS = 16
_NW = _NC * _NS
_C = _N // _NW
_WE = 8192
_NWIN = _C // _WE
_CAP = 4096

_PER_W = _MAX_FLIPS // _NW
_CHUNK = 128
_NCHUNK = _PER_W // _CHUNK

_SC_PARAMS = dict(
    mesh=plsc.VectorSubcoreMesh(core_axis_name="c", subcore_axis_name="s"),
    compiler_params=pltpu.CompilerParams(needs_layout_passes=False),
)


def _wid():
  return lax.axis_index("s") * _NC + lax.axis_index("c")


def _hist_body(keys_hbm, hist_out, hist_v, kwin, sem):
  del sem
  base = _wid() * _C
  z = jnp.zeros((16,), jnp.int32)

  def zero(t, c):
    hist_v[pl.ds(t * 16, 16)] = z
    return c

  lax.fori_loop(0, _NB // 16, zero, 0)

  def win(w, c):
    pltpu.sync_copy(keys_hbm.at[pl.ds(base + w * _WE, _WE)], kwin)

    def vreg(t, cc):
      d = lax.shift_right_logical(kwin[pl.ds(t * 16, 16)], 16)
      cnt, lastm = plsc.scan_count(d)
      plsc.addupdate_scatter(hist_v, [d], cnt, mask=lastm)
      return cc

    lax.fori_loop(0, _WE // 16, vreg, c)
    return c

  lax.fori_loop(0, _NWIN, win, 0)
  pltpu.sync_copy(hist_v, hist_out.at[_wid()])


def _compact_body(keys_hbm, t_hbm, ck_out, cp_out,
                  ck_v, cp_v, kwin, tv, sem):
  del sem
  wid = _wid()
  base = wid * _C
  pltpu.sync_copy(t_hbm, tv)
  thresh = tv[...]
  zero16 = jnp.zeros((16,), jnp.int32)
  iota = lax.iota(jnp.int32, 16)
  neg1 = zero16 - 1

  def init(t, c):
    ck_v[pl.ds(t * 16, 16)] = neg1
    cp_v[pl.ds(t * 16, 16)] = zero16
    return c

  lax.fori_loop(0, _CAP // 16, init, 0)

  def win(w, cur):
    pltpu.sync_copy(keys_hbm.at[pl.ds(base + w * _WE, _WE)], kwin)

    def vreg(t, cc):
      k = kwin[pl.ds(t * 16, 16)]
      d = lax.shift_right_logical(k, 16)
      m = d <= thresh
      mi = jnp.where(m, 1, 0).astype(jnp.int32)
      csum = plsc.cumsum(mi)
      dst = cc + csum - 1
      ok = jnp.logical_and(m, dst < _CAP)
      plsc.store_scatter(ck_v, [dst], k, mask=ok)
      pos = base + w * _WE + t * 16 + iota
      plsc.store_scatter(cp_v, [dst], pos, mask=ok)
      return cc + jnp.sum(mi)

    return lax.fori_loop(0, _WE // 16, vreg, cur)

  lax.fori_loop(0, _NWIN, win, jnp.int32(0))
  pltpu.sync_copy(ck_v, ck_out.at[wid])
  pltpu.sync_copy(cp_v, cp_out.at[wid])


@functools.cache
def _get_round3_kernels():
  hist = pl.kernel(
      _hist_body,
      out_type=jax.ShapeDtypeStruct((_NW, _NB), jnp.int32),
      scratch_types=[
          pltpu.VMEM((_NB,), jnp.int32),
          pltpu.VMEM((_WE,), jnp.int32),
          pltpu.SemaphoreType.DMA,
      ],
      **_SC_PARAMS,
  )
  compact = pl.kernel(
      _compact_body,
      out_type=(jax.ShapeDtypeStruct((_NW, _CAP), jnp.int32),
                jax.ShapeDtypeStruct((_NW, _CAP), jnp.int32)),
      scratch_types=[
          pltpu.VMEM((_CAP,), jnp.int32),
          pltpu.VMEM((_CAP,), jnp.int32),
          pltpu.VMEM((_WE,), jnp.int32),
          pltpu.VMEM((16,), jnp.int32),
          pltpu.SemaphoreType.DMA,
      ],
      **_SC_PARAMS,
  )
  return hist, compact


def _flip_body(p_hbm, flip_hbm, x2_hbm, x_ref,
               p_v, i_v, val_v, flip_v, sem):
  wid = _wid()
  pltpu.sync_copy(p_hbm.at[wid], p_v)
  pltpu.sync_copy(flip_hbm.at[wid], flip_v)

  g1 = [
      pltpu.async_copy(x2_hbm.at[p_v.at[j]], i_v.at[j], sem)
      for j in range(_NCHUNK)
  ]
  for g in g1:
    g.wait()

  g2 = [
      pltpu.async_copy(x_ref.at[i_v.at[j]],
                       val_v.at[pl.ds(j * _CHUNK, _CHUNK)], sem)
      for j in range(_NCHUNK)
  ]
  for g in g2:
    g.wait()

  def body(t, carry):
    s = pl.ds(t * 16, 16)
    val_v[s] = val_v[s] * flip_v[s]
    return carry

  lax.fori_loop(0, _PER_W // 16, body, 0)

  sc = [
      pltpu.async_copy(val_v.at[pl.ds(j * _CHUNK, _CHUNK)],
                       x_ref.at[i_v.at[j]], sem)
      for j in range(_NCHUNK)
  ]
  for s in sc:
    s.wait()


@functools.cache
def _get_flip_kernel():
  return pl.kernel(
      _flip_body,
      scratch_types=[
          pltpu.VMEM((_NCHUNK, _CHUNK), jnp.int32),
          pltpu.VMEM((_NCHUNK, _CHUNK), jnp.int32),
          pltpu.VMEM((_PER_W,), jnp.float32),
          pltpu.VMEM((_PER_W,), jnp.float32),
          pltpu.SemaphoreType.DMA,
      ],
      **_SC_PARAMS,
  )


def kernel(x, key):
  shape = x.shape
  xf = x.ravel()
  key1, key2 = jr.split(key, 2)
  vals = jnp.array([-1, 1], dtype=xf.dtype)
  flip = jr.choice(key2, vals, (_MAX_FLIPS,))

  perm = jnp.arange(_N, dtype=jnp.int32)
  k = key1
  subkeys = []
  for _ in range(3):
    k, sk = jr.split(k)
    subkeys.append(sk)
  for sk in subkeys[:2]:
    bits = jr.bits(sk, (_N,), jnp.uint32)
    _, perm = lax.sort_key_val(bits, perm)

  bits3 = lax.bitcast_convert_type(
      jr.bits(subkeys[2], (_N,), jnp.uint32), jnp.int32)
  hist, compact = _get_round3_kernels()
  h = hist(bits3)
  counts = h.sum(axis=0, dtype=jnp.int32)
  cum = jnp.cumsum(counts, dtype=jnp.int32)
  thresh = jnp.argmax(cum >= _MAX_FLIPS).astype(jnp.int32)
  ck, cp = compact(bits3, jnp.full((16,), thresh, jnp.int32))
  cku = lax.bitcast_convert_type(ck.reshape(-1), jnp.uint32)
  _, p_sorted = lax.sort_key_val(cku, cp.reshape(-1))
  p = p_sorted[:_MAX_FLIPS]

  p3 = p.reshape(_NW, _NCHUNK, _CHUNK)
  flip2 = flip.reshape(_NW, _PER_W)
  ref = jax.new_ref(xf)
  _get_flip_kernel()(p3, flip2, perm, ref)
  return ref[...].reshape(shape)
